# Initial kernel scaffold; baseline (speedup 1.0000x reference)
import jax
import jax.numpy as jnp
from jax.experimental import pallas as pl


def kernel(x, table, W1, b1, W2, b2):
    raise NotImplementedError("write your pallas kernel here")


# speedup vs baseline: 2.4244x; 2.4244x over previous
import functools

import jax
import jax.numpy as jnp
from jax import lax
from jax.experimental import pallas as pl
from jax.experimental.pallas import tpu as pltpu
from jax.experimental.pallas import tpu_sc as plsc

D = 32
B = 4096
L = 200
HALF = 100
NC = 2
NS = 16
NW = NC * NS
BPW = B // NW
NBUF = 4
GROUPS = BPW // NBUF


_sc_mesh = plsc.VectorSubcoreMesh(core_axis_name="c", subcore_axis_name="s")


@functools.partial(
    pl.kernel,
    out_type=jax.ShapeDtypeStruct((B, D), jnp.float32),
    mesh=_sc_mesh,
    scratch_types=[
        pltpu.VMEM((BPW * 2, HALF), jnp.int32),
        pltpu.VMEM((NBUF, L, D), jnp.float32),
        pltpu.VMEM((BPW, D), jnp.float32),
    ] + [pltpu.SemaphoreType.DMA] * NBUF,
    compiler_params=pltpu.CompilerParams(use_tc_tiling_on_sc=False),
)
def _gather_pool(x_hbm, table_hbm, pooled_hbm, idx_v, buf_v, pooled_v, *sems):
    wid = lax.axis_index("s") * NC + lax.axis_index("c")
    chunk_base = wid * (BPW * 2)

    pltpu.sync_copy(x_hbm.at[pl.ds(chunk_base, BPW * 2), :], idx_v)

    def start_row(r, slot):
        pltpu.make_async_copy(
            table_hbm.at[idx_v.at[2 * r]],
            buf_v.at[slot, pl.ds(0, HALF)], sems[slot]).start()
        pltpu.make_async_copy(
            table_hbm.at[idx_v.at[2 * r + 1]],
            buf_v.at[slot, pl.ds(HALF, HALF)], sems[slot]).start()

    def wait_row(slot):
        for h in (0, 1):
            pltpu.make_async_copy(
                table_hbm.at[idx_v.at[h]],
                buf_v.at[slot, pl.ds(h * HALF, HALF)], sems[slot]).wait()

    for i in range(NBUF):
        start_row(i, i)

    @pl.loop(0, GROUPS)
    def _group(g):
        for i in range(NBUF):
            r = g * NBUF + i
            wait_row(i)

            zero = jnp.zeros((16,), jnp.float32)

            @pl.loop(0, L // 4, init_carry=(zero, zero, zero, zero))
            def reduce4(jj, carry):
                a0, a1, c0, c1 = carry
                j = jj * 4
                a0 = a0 + buf_v[i, j, pl.ds(0, 16)]
                a1 = a1 + buf_v[i, j, pl.ds(16, 16)]
                c0 = c0 + buf_v[i, j + 1, pl.ds(0, 16)]
                c1 = c1 + buf_v[i, j + 1, pl.ds(16, 16)]
                a0 = a0 + buf_v[i, j + 2, pl.ds(0, 16)]
                a1 = a1 + buf_v[i, j + 2, pl.ds(16, 16)]
                c0 = c0 + buf_v[i, j + 3, pl.ds(0, 16)]
                c1 = c1 + buf_v[i, j + 3, pl.ds(16, 16)]
                return a0, a1, c0, c1

            a0, a1, c0, c1 = reduce4
            pooled_v[r, pl.ds(0, 16)] = a0 + c0
            pooled_v[r, pl.ds(16, 16)] = a1 + c1

            @pl.when(g < GROUPS - 1)
            def _refill():
                start_row(r + NBUF, i)

    pltpu.sync_copy(pooled_v, pooled_hbm.at[pl.ds(wid * BPW, BPW), :])


def _mlp_body(p_ref, w1_ref, b1_ref, w2t_ref, b2_ref, o_ref):
    p = p_ref[...] * (1.0 / L)
    h = jnp.maximum(
        jnp.dot(p, w1_ref[...], preferred_element_type=jnp.float32)
        + b1_ref[...], 0.0)
    o = jnp.sum(h * w2t_ref[...], axis=1, keepdims=True) + b2_ref[...]
    o_ref[...] = 1.0 / (1.0 + jnp.exp(-o))


def kernel(x, table, W1, b1, W2, b2):
    xi = x.astype(jnp.int32).reshape(B * 2, HALF)
    pooled = _gather_pool(xi, table)
    out = pl.pallas_call(
        _mlp_body,
        out_shape=jax.ShapeDtypeStruct((B, 1), jnp.float32),
    )(pooled, W1, b1.reshape(1, 16), W2.reshape(1, 16), b2.reshape(1, 1))
    return out
